# baseline (speedup 1.0000x reference)
import functools

import jax
import jax.numpy as jnp
from jax import lax
from jax.experimental import pallas as pl
from jax.experimental.pallas import tpu as pltpu, tpu_sc as plsc

_INFO = plsc.get_sparse_core_info()
_NC, _NS = _INFO.num_cores, _INFO.num_subcores
_NW = _NC * _NS
_IW = 128
_K = 4
_CHUNK = _IW * _K


@functools.partial(jax.jit, static_argnums=(2,))
def _sc_gather(idx2d, table, num_chunks):
    D = table.shape[1]
    B = idx2d.shape[0] * idx2d.shape[1]
    rows_per_w = num_chunks * _K
    mesh = plsc.VectorSubcoreMesh(core_axis_name="c", subcore_axis_name="s")

    @functools.partial(
        pl.kernel,
        out_type=jax.ShapeDtypeStruct((B, D), jnp.float32),
        mesh=mesh,
        scratch_types=[
            pltpu.VMEM((_K, _IW), jnp.int32),
            pltpu.VMEM((_CHUNK, D), jnp.float32),
            pltpu.SemaphoreType.DMA,
        ],
        compiler_params=pltpu.CompilerParams(use_tc_tiling_on_sc=False),
    )
    def k(idx_hbm, table_hbm, out_hbm, idx_v, rows_v, sem):
        wid = lax.axis_index("s") * _NC + lax.axis_index("c")
        row_base = wid * rows_per_w

        def body(g, carry):
            irow = row_base + g * _K
            pltpu.sync_copy(idx_hbm.at[pl.ds(irow, _K)], idx_v)
            handles = [
                pltpu.async_copy(
                    table_hbm.at[idx_v.at[j]],
                    rows_v.at[pl.ds(j * _IW, _IW)],
                    sem,
                )
                for j in range(_K)
            ]
            for h in handles:
                h.wait()
            pltpu.sync_copy(rows_v, out_hbm.at[pl.ds(irow * _IW, _CHUNK)])
            return carry

        lax.fori_loop(0, num_chunks, body, 0)

    return k(idx2d, table)


def kernel(location_x, loc_table, time_table, user_table):
    orig_shape = location_x.shape
    idx_flat = location_x.reshape(-1).astype(jnp.int32)
    B = idx_flat.shape[0]
    b_per_w = B // _NW
    num_chunks = b_per_w // _CHUNK
    idx2d = idx_flat.reshape(B // _IW, _IW)
    out = _sc_gather(idx2d, loc_table, num_chunks)
    loc_emb = out.reshape(orig_shape + (loc_table.shape[1],))
    return (loc_emb, time_table, user_table)


# speedup vs baseline: 1.0326x; 1.0326x over previous
import functools

import jax
import jax.numpy as jnp
from jax import lax
from jax.experimental import pallas as pl
from jax.experimental.pallas import tpu as pltpu, tpu_sc as plsc

_INFO = plsc.get_sparse_core_info()
_NC, _NS = _INFO.num_cores, _INFO.num_subcores
_NW = _NC * _NS
_IW = 128
_K = 4
_CHUNK = _IW * _K


@functools.partial(jax.jit, static_argnums=(2,))
def _sc_gather(idx2d, table, num_chunks):
    D = table.shape[1]
    B = idx2d.shape[0] * idx2d.shape[1]
    rows_per_w = num_chunks * _K
    mesh = plsc.VectorSubcoreMesh(core_axis_name="c", subcore_axis_name="s")

    @functools.partial(
        pl.kernel,
        out_type=jax.ShapeDtypeStruct((B, D), jnp.float32),
        mesh=mesh,
        scratch_types=[
            pltpu.VMEM((_K, _IW), jnp.int32),
            pltpu.VMEM((_K, _IW), jnp.int32),
            pltpu.VMEM((_CHUNK, D), jnp.float32),
            pltpu.VMEM((_CHUNK, D), jnp.float32),
            pltpu.SemaphoreType.DMA,
            pltpu.SemaphoreType.DMA,
        ],
        compiler_params=pltpu.CompilerParams(use_tc_tiling_on_sc=False),
    )
    def k(idx_hbm, table_hbm, out_hbm, idx_a, idx_b, rows_a, rows_b, sem_a, sem_b):
        wid = lax.axis_index("s") * _NC + lax.axis_index("c")
        row_base = wid * rows_per_w

        def stage(idx_v, rows_v, sem, irow):
            pltpu.sync_copy(idx_hbm.at[pl.ds(irow, _K)], idx_v)
            for j in range(_K):
                pltpu.async_copy(
                    table_hbm.at[idx_v.at[j]],
                    rows_v.at[pl.ds(j * _IW, _IW)],
                    sem,
                )

        def drain(rows_v, sem, irow):
            for j in range(_K):
                pltpu.make_async_copy(
                    table_hbm.at[pl.ds(0, _IW)],
                    rows_v.at[pl.ds(j * _IW, _IW)],
                    sem,
                ).wait()
            pltpu.sync_copy(rows_v, out_hbm.at[pl.ds(irow * _IW, _CHUNK)])

        stage(idx_a, rows_a, sem_a, row_base)

        def body(p, carry):
            irow_a = row_base + 2 * p * _K
            stage(idx_b, rows_b, sem_b, irow_a + _K)
            drain(rows_a, sem_a, irow_a)

            @pl.when(p < num_chunks // 2 - 1)
            def _():
                stage(idx_a, rows_a, sem_a, irow_a + 2 * _K)

            drain(rows_b, sem_b, irow_a + _K)
            return carry

        lax.fori_loop(0, num_chunks // 2, body, 0)

    return k(idx2d, table)


def kernel(location_x, loc_table, time_table, user_table):
    orig_shape = location_x.shape
    idx_flat = location_x.reshape(-1).astype(jnp.int32)
    B = idx_flat.shape[0]
    b_per_w = B // _NW
    num_chunks = b_per_w // _CHUNK
    idx2d = idx_flat.reshape(B // _IW, _IW)
    out = _sc_gather(idx2d, loc_table, num_chunks)
    loc_emb = out.reshape(orig_shape + (loc_table.shape[1],))
    return (loc_emb, time_table, user_table)
